# Initial kernel scaffold; baseline (speedup 1.0000x reference)
import jax
import jax.numpy as jnp
from jax.experimental import pallas as pl


def kernel(points, global_colors, local_colors, alpha):
    raise NotImplementedError("write your pallas kernel here")


# speedup vs baseline: 5.8162x; 5.8162x over previous
import functools

import jax
import jax.numpy as jnp
from jax import lax
from jax.experimental import pallas as pl
from jax.experimental.pallas import tpu as pltpu
from jax.experimental.pallas import tpu_sc as plsc

_NC = 2
_NS = 16
_L = 16
_NW = _NC * _NS

_RES = 256
_CHUNK = 4096


def _build_sc_call(n_points):
    pts_per_tile = n_points // _NW
    chunks = pts_per_tile // _CHUNK
    c5 = _CHUNK * 5
    c6 = _CHUNK * 6
    tbl_words = _RES * 3

    mesh = plsc.VectorSubcoreMesh(core_axis_name="c", subcore_axis_name="s")

    @functools.partial(
        pl.kernel,
        mesh=mesh,
        out_type=jax.ShapeDtypeStruct((n_points * 6,), jnp.float32),
        compiler_params=pltpu.CompilerParams(needs_layout_passes=False),
        scratch_types=[
            pltpu.VMEM((c5,), jnp.float32),
            pltpu.VMEM((c6,), jnp.float32),
            pltpu.VMEM((tbl_words,), jnp.float32),
            pltpu.VMEM((tbl_words,), jnp.float32),
            pltpu.VMEM((tbl_words,), jnp.float32),
            pltpu.VMEM((_L,), jnp.float32),
        ],
    )
    def sc_kernel(pts_hbm, g_hbm, l_hbm, a_hbm, out_hbm,
                  in_buf, out_buf, tbl, gbuf, lbuf, abuf):
        wid = lax.axis_index("s") * _NC + lax.axis_index("c")

        pltpu.sync_copy(g_hbm, gbuf)
        pltpu.sync_copy(l_hbm, lbuf)
        pltpu.sync_copy(a_hbm, abuf)
        av = abuf[...]
        bv = 1.0 - av

        def blend_body(i, _):
            sl = pl.ds(i * _L, _L)
            gi = jnp.clip(gbuf[sl], 0.0, 1.0)
            li = jnp.clip(lbuf[sl], 0.0, 1.0)
            tbl[sl] = av * gi + bv * li
            return _

        lax.fori_loop(0, tbl_words // _L, blend_body, None)

        iota = lax.iota(jnp.int32, _L)
        i5 = iota * 5
        i6 = iota * 6
        tile_base = wid * pts_per_tile

        def point_body(it, _):
            in_off = it * (5 * _L)
            out_off = it * (6 * _L)
            xi = i5 + in_off
            x = plsc.load_gather(in_buf, [xi])
            y = plsc.load_gather(in_buf, [xi + 1])
            z = plsc.load_gather(in_buf, [xi + 2])
            d = plsc.load_gather(in_buf, [xi + 3])
            di = d.astype(jnp.int32)
            ci = jnp.clip((di * (_RES - 1)) // 255, 0, _RES - 1)
            t0 = ci * 3
            r = plsc.load_gather(tbl, [t0])
            g = plsc.load_gather(tbl, [t0 + 1])
            b = plsc.load_gather(tbl, [t0 + 2])
            oi = i6 + out_off
            plsc.store_scatter(out_buf, [oi], av * x + bv * x)
            plsc.store_scatter(out_buf, [oi + 1], av * y + bv * y)
            plsc.store_scatter(out_buf, [oi + 2], av * z + bv * z)
            plsc.store_scatter(out_buf, [oi + 3], r)
            plsc.store_scatter(out_buf, [oi + 4], g)
            plsc.store_scatter(out_buf, [oi + 5], b)
            return _

        def chunk_body(c, _):
            base = tile_base + c * _CHUNK
            pltpu.sync_copy(pts_hbm.at[pl.ds(base * 5, c5)], in_buf)
            lax.fori_loop(0, _CHUNK // _L, point_body, None)
            pltpu.sync_copy(out_buf, out_hbm.at[pl.ds(base * 6, c6)])
            return _

        lax.fori_loop(0, chunks, chunk_body, None)

    return sc_kernel


def kernel(points, global_colors, local_colors, alpha):
    n = points.shape[0]
    pts_flat = points.reshape(-1)
    g_flat = global_colors.reshape(-1)
    l_flat = local_colors.reshape(-1)
    alpha16 = jnp.broadcast_to(alpha.reshape(1), (_L,))
    out_flat = _build_sc_call(n)(pts_flat, g_flat, l_flat, alpha16)
    return out_flat.reshape(n, 6)


# speedup vs baseline: 6.4176x; 1.1034x over previous
import functools

import jax
import jax.numpy as jnp
from jax import lax
from jax.experimental import pallas as pl
from jax.experimental.pallas import tpu as pltpu
from jax.experimental.pallas import tpu_sc as plsc

_NC = 2
_NS = 16
_L = 16
_NW = _NC * _NS

_RES = 256
_CHUNK = 4096
_UNROLL = 8


def _build_sc_call(n_points):
    pts_per_tile = n_points // _NW
    chunks = pts_per_tile // _CHUNK
    c5 = _CHUNK * 5
    c6 = _CHUNK * 6
    tbl_words = _RES * 3

    mesh = plsc.VectorSubcoreMesh(core_axis_name="c", subcore_axis_name="s")

    @functools.partial(
        pl.kernel,
        mesh=mesh,
        out_type=jax.ShapeDtypeStruct((n_points * 6,), jnp.float32),
        compiler_params=pltpu.CompilerParams(needs_layout_passes=False),
        scratch_types=[
            pltpu.VMEM((c5,), jnp.float32),
            pltpu.VMEM((c5,), jnp.float32),
            pltpu.VMEM((c6,), jnp.float32),
            pltpu.VMEM((c6,), jnp.float32),
            pltpu.VMEM((tbl_words,), jnp.float32),
            pltpu.VMEM((tbl_words,), jnp.float32),
            pltpu.VMEM((tbl_words,), jnp.float32),
            pltpu.VMEM((_L,), jnp.float32),
            pltpu.SemaphoreType.DMA,
            pltpu.SemaphoreType.DMA,
            pltpu.SemaphoreType.DMA,
            pltpu.SemaphoreType.DMA,
        ],
    )
    def sc_kernel(pts_hbm, g_hbm, l_hbm, a_hbm, out_hbm,
                  in_a, in_b, out_a, out_b, tbl, gbuf, lbuf, abuf,
                  isem_a, isem_b, osem_a, osem_b):
        wid = lax.axis_index("s") * _NC + lax.axis_index("c")
        tile_base = wid * pts_per_tile

        in_bufs = (in_a, in_b)
        out_bufs = (out_a, out_b)
        isems = (isem_a, isem_b)
        osems = (osem_a, osem_b)

        def start_in(c):
            base = tile_base + c * _CHUNK
            return pltpu.async_copy(
                pts_hbm.at[pl.ds(base * 5, c5)], in_bufs[c % 2], isems[c % 2])

        def start_out(c):
            base = tile_base + c * _CHUNK
            return pltpu.async_copy(
                out_bufs[c % 2], out_hbm.at[pl.ds(base * 6, c6)], osems[c % 2])

        in_dma = [None, None]
        out_dma = [None, None]
        in_dma[0] = start_in(0)

        pltpu.sync_copy(g_hbm, gbuf)
        pltpu.sync_copy(l_hbm, lbuf)
        pltpu.sync_copy(a_hbm, abuf)
        av = abuf[...]
        bv = 1.0 - av

        def blend_body(i, _):
            sl = pl.ds(i * _L, _L)
            gi = jnp.clip(gbuf[sl], 0.0, 1.0)
            li = jnp.clip(lbuf[sl], 0.0, 1.0)
            tbl[sl] = av * gi + bv * li
            return _

        lax.fori_loop(0, tbl_words // _L, blend_body, None)

        iota = lax.iota(jnp.int32, _L)
        i5 = iota * 5
        i6 = iota * 6

        def compute_chunk(in_buf, out_buf):
            @plsc.parallel_loop(0, _CHUNK // _L, unroll=_UNROLL)
            def point_body(it):
                xi = i5 + it * (5 * _L)
                x = plsc.load_gather(in_buf, [xi])
                y = plsc.load_gather(in_buf, [xi + 1])
                z = plsc.load_gather(in_buf, [xi + 2])
                d = plsc.load_gather(in_buf, [xi + 3])
                ci = jnp.clip(d.astype(jnp.int32), 0, _RES - 1)
                t0 = ci * 3
                r = plsc.load_gather(tbl, [t0])
                g = plsc.load_gather(tbl, [t0 + 1])
                b = plsc.load_gather(tbl, [t0 + 2])
                oi = i6 + it * (6 * _L)
                plsc.store_scatter(out_buf, [oi], av * x + bv * x)
                plsc.store_scatter(out_buf, [oi + 1], av * y + bv * y)
                plsc.store_scatter(out_buf, [oi + 2], av * z + bv * z)
                plsc.store_scatter(out_buf, [oi + 3], r)
                plsc.store_scatter(out_buf, [oi + 4], g)
                plsc.store_scatter(out_buf, [oi + 5], b)

        for c in range(chunks):
            cur = c % 2
            in_dma[cur].wait()
            if c + 1 < chunks:
                in_dma[(c + 1) % 2] = start_in(c + 1)
            if c >= 2:
                out_dma[cur].wait()
            compute_chunk(in_bufs[cur], out_bufs[cur])
            out_dma[cur] = start_out(c)

        if chunks >= 2:
            out_dma[(chunks - 2) % 2].wait()
        out_dma[(chunks - 1) % 2].wait()

    return sc_kernel


def kernel(points, global_colors, local_colors, alpha):
    n = points.shape[0]
    pts_flat = points.reshape(-1)
    g_flat = global_colors.reshape(-1)
    l_flat = local_colors.reshape(-1)
    alpha16 = jnp.broadcast_to(alpha.reshape(1), (_L,))
    out_flat = _build_sc_call(n)(pts_flat, g_flat, l_flat, alpha16)
    return out_flat.reshape(n, 6)


# speedup vs baseline: 192.0450x; 29.9249x over previous
import functools

import jax
import jax.numpy as jnp
from jax import lax
from jax.experimental import pallas as pl
from jax.experimental.pallas import tpu as pltpu
from jax.experimental.pallas import tpu_sc as plsc

_NC = 2
_NS = 16
_L = 16
_NW = _NC * _NS

_RES = 256
_CHUNK = 8192
_UNROLL = 8


def _build_sc_call(n_points):
    pts_per_tile = n_points // _NW
    chunks = pts_per_tile // _CHUNK
    tbl_words = _RES * 3

    mesh = plsc.VectorSubcoreMesh(core_axis_name="c", subcore_axis_name="s")

    buf = pltpu.VMEM((_CHUNK,), jnp.float32)

    @functools.partial(
        pl.kernel,
        mesh=mesh,
        out_type=jax.ShapeDtypeStruct((6, n_points), jnp.float32),
        compiler_params=pltpu.CompilerParams(
            needs_layout_passes=False, use_tc_tiling_on_sc=True),
        scratch_types=[
            [buf] * 7,
            [buf] * 7,
            pltpu.VMEM((tbl_words,), jnp.float32),
            pltpu.VMEM((tbl_words,), jnp.float32),
            pltpu.VMEM((tbl_words,), jnp.float32),
            pltpu.VMEM((_L,), jnp.float32),
            pltpu.SemaphoreType.DMA,
            pltpu.SemaphoreType.DMA,
            pltpu.SemaphoreType.DMA,
            pltpu.SemaphoreType.DMA,
        ],
    )
    def sc_kernel(pts_hbm, g_hbm, l_hbm, a_hbm, out_hbm,
                  set_a, set_b, tbl, gbuf, lbuf, abuf,
                  isem_a, isem_b, osem_a, osem_b):
        wid = lax.axis_index("s") * _NC + lax.axis_index("c")
        tile_base = wid * pts_per_tile

        sets = (set_a, set_b)
        isems = (isem_a, isem_b)
        osems = (osem_a, osem_b)

        def start_in(c):
            s = c % 2
            base = tile_base + c * _CHUNK
            sl = pl.ds(base, _CHUNK)
            return [pltpu.async_copy(pts_hbm.at[f, sl], sets[s][f], isems[s])
                    for f in range(4)]

        def start_out(c):
            s = c % 2
            base = tile_base + c * _CHUNK
            sl = pl.ds(base, _CHUNK)
            return [pltpu.async_copy(sets[s][f], out_hbm.at[f, sl], osems[s])
                    for f in range(3)] + \
                   [pltpu.async_copy(sets[s][f + 4], out_hbm.at[f + 3, sl],
                                     osems[s])
                    for f in range(3)]

        in_dma = [None, None]
        out_dma = [None, None]
        in_dma[0] = start_in(0)

        pltpu.sync_copy(g_hbm, gbuf)
        pltpu.sync_copy(l_hbm, lbuf)
        pltpu.sync_copy(a_hbm, abuf)
        av = abuf[...]
        bv = 1.0 - av

        def blend_body(i, _):
            sl = pl.ds(i * _L, _L)
            gi = jnp.clip(gbuf[sl], 0.0, 1.0)
            li = jnp.clip(lbuf[sl], 0.0, 1.0)
            tbl[sl] = av * gi + bv * li
            return _

        lax.fori_loop(0, tbl_words // _L, blend_body, None)

        def compute_chunk(s):
            db, rb, gb, bb = sets[s][3], sets[s][4], sets[s][5], sets[s][6]

            @plsc.parallel_loop(0, _CHUNK // _L, unroll=_UNROLL)
            def point_body(it):
                sl = pl.ds(it * _L, _L)
                ci = jnp.clip(db[sl].astype(jnp.int32), 0, _RES - 1)
                t0 = ci * 3
                rb[sl] = plsc.load_gather(tbl, [t0])
                gb[sl] = plsc.load_gather(tbl, [t0 + 1])
                bb[sl] = plsc.load_gather(tbl, [t0 + 2])

        for c in range(chunks):
            cur = c % 2
            if c + 1 < chunks:
                if c >= 1:
                    for d in out_dma[1 - cur]:
                        d.wait()
                in_dma[1 - cur] = start_in(c + 1)
            for d in in_dma[cur]:
                d.wait()
            compute_chunk(cur)
            out_dma[cur] = start_out(c)

        if chunks >= 2:
            for d in out_dma[chunks % 2]:
                d.wait()
        for d in out_dma[(chunks - 1) % 2]:
            d.wait()

    return sc_kernel


def kernel(points, global_colors, local_colors, alpha):
    n = points.shape[0]
    pts_t = points.T
    g_flat = global_colors.reshape(-1)
    l_flat = local_colors.reshape(-1)
    alpha16 = jnp.broadcast_to(alpha.reshape(1), (_L,))
    out_t = _build_sc_call(n)(pts_t, g_flat, l_flat, alpha16)
    return out_t.T
